# baseline (speedup 1.0000x reference)
import jax
import jax.numpy as jnp
from jax import lax
from jax.experimental import pallas as pl
from jax.experimental.pallas import tpu as pltpu
from jax.experimental.pallas import tpu_sc as plsc

_B = 100000
_NL = 4
_DIMS = (16, 32, 32, 48)
_OFFS = (0, 16, 48, 80)
_OUT_D = 128
_NC, _NS = 2, 16
_NW = _NC * _NS
_SG = 128
_GPC = 2
_C = _SG * _GPC
_NSUB = -(-_B // _SG)
_BPAD = _NSUB * _SG
_K = _B // _C
_TAIL = _B - _K * _C
_NKMAX = 13
_PMAX = -(-_NKMAX // 2)
_NSPAN = _GPC * _NKMAX
_TAILW = _NW - 1


def _body(cl3, t0, t1, t2, t3, out,
          allidx, a0, a1, a2, a3, b0, b1, b2, b3,
          gsem, wsem0, wsem1):
    tabs = (t0, t1, t2, t3)
    rows = ((a0, a1, a2, a3), (b0, b1, b2, b3))
    wsems = (wsem0, wsem1)
    wid = lax.axis_index("s") * _NC + lax.axis_index("c")
    base_k = 12 * wid + jnp.minimum(wid, 6)
    nk = jnp.where(wid < 6, 13, 12)

    def out_slc(s, l):
        return out.at[pl.ds(s, _C), pl.ds(_OFFS[l], _DIMS[l])]

    fb = pl.multiple_of(_GPC * base_k, _GPC)
    pltpu.sync_copy(cl3.at[:, pl.ds(fb, _NSPAN), :], allidx)

    def gather_all(i, rowset):
        return [
            pltpu.async_copy(tabs[l].at[allidx.at[l, i * _GPC + j]],
                             rowset[l].at[pl.ds(j * _SG, _SG)], gsem)
            for l in range(_NL) for j in range(_GPC)
        ]

    def chunk(i, b):
        s = pl.multiple_of((base_k + i) * _C, _C)

        @pl.when(i >= 2)
        def _drain():
            for l in range(_NL):
                pltpu.make_async_copy(rows[b][l], out_slc(s, l),
                                      wsems[b]).wait()

        gcps = gather_all(i, rows[b])
        for cp in gcps:
            cp.wait()

        for l in range(_NL):
            pltpu.async_copy(rows[b][l], out_slc(s, l), wsems[b])

    def pair(p, carry):
        for b in (0, 1):
            i = 2 * p + b

            @pl.when(i < nk)
            def _():
                chunk(i, b)

        return carry

    lax.fori_loop(0, _PMAX, pair, 0)

    for b in (0, 1):
        for l in range(_NL):
            pltpu.make_async_copy(rows[b][l], out_slc(0, l),
                                  wsems[b]).wait()

    @pl.when(wid == _TAILW)
    def _tail():
        gcps = gather_all(12, rows[0])
        for cp in gcps:
            cp.wait()
        wcps = [
            pltpu.async_copy(
                rows[0][l].at[pl.ds(0, _TAIL)],
                out.at[pl.ds(_K * _C, _TAIL), pl.ds(_OFFS[l], _DIMS[l])],
                wsems[0])
            for l in range(_NL)
        ]
        for cp in wcps:
            cp.wait()


@jax.jit
def kernel(code_levels, table_0, table_1, table_2, table_3):
    cl_t = code_levels.T.astype(jnp.int32)
    cl3 = jnp.pad(cl_t, ((0, 0), (0, _BPAD - _B))).reshape(_NL, _NSUB, _SG)
    run = pl.kernel(
        _body,
        out_type=jax.ShapeDtypeStruct((_B, _OUT_D), jnp.float32),
        mesh=plsc.VectorSubcoreMesh(core_axis_name="c", subcore_axis_name="s",
                                    num_cores=_NC, num_subcores=_NS),
        scratch_types=[
            pltpu.VMEM((_NL, _NSPAN, _SG), jnp.int32),
            pltpu.VMEM((_C, _DIMS[0]), jnp.float32),
            pltpu.VMEM((_C, _DIMS[1]), jnp.float32),
            pltpu.VMEM((_C, _DIMS[2]), jnp.float32),
            pltpu.VMEM((_C, _DIMS[3]), jnp.float32),
            pltpu.VMEM((_C, _DIMS[0]), jnp.float32),
            pltpu.VMEM((_C, _DIMS[1]), jnp.float32),
            pltpu.VMEM((_C, _DIMS[2]), jnp.float32),
            pltpu.VMEM((_C, _DIMS[3]), jnp.float32),
            pltpu.SemaphoreType.DMA,
            pltpu.SemaphoreType.DMA,
            pltpu.SemaphoreType.DMA,
        ],
        compiler_params=pltpu.CompilerParams(use_tc_tiling_on_sc=False),
    )
    return run(cl3, table_0, table_1, table_2, table_3)
